# baseline (speedup 1.0000x reference)
import functools

import jax
import jax.numpy as jnp
from jax import lax
from jax.experimental import pallas as pl
from jax.experimental.pallas import tpu as pltpu
from jax.experimental.pallas import tpu_sc as plsc

N = 10000
NP = 10240
D = 128
H = 16
NC = 2
NS = 16
NW = NC * NS
E = 320000
EP = E // NW
CH = 128
G = 8
K1 = 80
NG = K1 // G
EPP = K1 * CH
RPT = NP // NS
U = 5

_mesh = plsc.VectorSubcoreMesh(core_axis_name="c", subcore_axis_name="s")
_params = pltpu.CompilerParams(needs_layout_passes=False,
                               use_tc_tiling_on_sc=False)


def _zero_1d(ref, n):
    def zbody(i, carry):
        ref[pl.ds(i * 16, 16)] = jnp.zeros((16,), jnp.float32)
        return carry

    lax.fori_loop(0, n // 16, zbody, 0)


def _spmem_reduce_write(acc_v, shared, red, tmp, out_hbm, cid, sid):
    pltpu.sync_copy(acc_v, shared.at[sid])
    plsc.subcore_barrier()
    sl = pl.ds(sid * RPT, RPT)
    pltpu.sync_copy(shared.at[0].at[sl], red)
    for t in range(1, NS):
        pltpu.sync_copy(shared.at[t].at[sl], tmp)

        def abody(i, c2):
            red[pl.ds(i * 16, 16)] = (red[pl.ds(i * 16, 16)]
                                      + tmp[pl.ds(i * 16, 16)])
            return c2

        lax.fori_loop(0, RPT // 16, abody, 0)
    pltpu.sync_copy(red, out_hbm.at[pl.ds(cid * NP + sid * RPT, RPT)])


@functools.partial(
    pl.kernel,
    out_type=jax.ShapeDtypeStruct((NC * NP,), jnp.float32),
    mesh=_mesh,
    compiler_params=_params,
    scratch_types=[
        pltpu.VMEM((EP,), jnp.int32),
        pltpu.VMEM((NP,), jnp.float32),
        pltpu.VMEM_SHARED((NS, NP), jnp.float32),
        pltpu.VMEM((RPT,), jnp.float32),
        pltpu.VMEM((RPT,), jnp.float32),
        pltpu.SemaphoreType.DMA,
    ],
)
def _deg_kernel(dst_hbm, out_hbm, dst_v, acc_v, shared, red, tmp, sem):
    del sem
    cid = lax.axis_index("c")
    sid = lax.axis_index("s")
    wid = sid * NC + cid
    pltpu.sync_copy(dst_hbm.at[wid], dst_v)
    _zero_1d(acc_v, NP)
    ones = jnp.ones((16,), jnp.float32)

    def cbody(j, carry):
        for u in range(U):
            d16 = dst_v[pl.ds((j * U + u) * 16, 16)]
            plsc.addupdate_scatter(acc_v, [d16], ones)
        return carry

    lax.fori_loop(0, EP // (16 * U), cbody, 0)
    _spmem_reduce_write(acc_v, shared, red, tmp, out_hbm, cid, sid)


@functools.partial(
    pl.kernel,
    out_type=jax.ShapeDtypeStruct((NC, NP, H), jnp.float32),
    mesh=_mesh,
    compiler_params=_params,
    scratch_types=[
        pltpu.VMEM((K1, CH), jnp.int32),
        pltpu.VMEM((K1, CH), jnp.int32),
        pltpu.VMEM((3, G, CH, H), jnp.float32),
        pltpu.VMEM((RPT, H), jnp.float32),
        pltpu.VMEM_SHARED((NP, H), jnp.float32),
        pltpu.SemaphoreType.DMA,
        pltpu.SemaphoreType.DMA,
    ],
)
def _agg1_kernel(hs_hbm, src_hbm, dst_hbm, out_hbm, si, di, rows, zbuf,
                 acc_sh, sem_g, sem_s):
    cid = lax.axis_index("c")
    sid = lax.axis_index("s")
    wid = sid * NC + cid
    pltpu.sync_copy(src_hbm.at[wid], si)
    pltpu.sync_copy(dst_hbm.at[wid], di)

    def zb(i, carry):
        zbuf[i, :] = jnp.zeros((H,), jnp.float32)
        return carry

    lax.fori_loop(0, RPT, zb, 0)
    pltpu.sync_copy(zbuf, acc_sh.at[pl.ds(sid * RPT, RPT)])
    plsc.subcore_barrier()

    for b in range(G):
        pltpu.async_copy(hs_hbm.at[si.at[b]], rows.at[0, b], sem_g)

    def body(g, carry):
        b_cur = lax.rem(g, 3)
        b_nxt = lax.rem(g + 1, 3)

        @pl.when(g >= 2)
        def _drain():
            for b in range(G):
                pltpu.make_async_copy(rows.at[b_nxt, b],
                                      acc_sh.at[di.at[(g - 2) * G + b]],
                                      sem_s).wait()

        @pl.when(g + 1 < NG)
        def _fire():
            for b in range(G):
                pltpu.async_copy(hs_hbm.at[si.at[(g + 1) * G + b]],
                                 rows.at[b_nxt, b], sem_g)

        for b in range(G):
            pltpu.make_async_copy(hs_hbm.at[si.at[g * G + b]],
                                  rows.at[b_cur, b], sem_g).wait()
        for b in range(G):
            pltpu.async_copy(rows.at[b_cur, b], acc_sh.at[di.at[g * G + b]],
                             sem_s, add=True)
        return carry

    lax.fori_loop(0, NG, body, 0)
    for g in (NG - 2, NG - 1):
        for b in range(G):
            pltpu.make_async_copy(rows.at[g % 3, b],
                                  acc_sh.at[di.at[g * G + b]], sem_s).wait()
    plsc.subcore_barrier()
    sl = pl.ds(sid * RPT, RPT)
    pltpu.sync_copy(acc_sh.at[sl], out_hbm.at[cid].at[sl])


@functools.partial(
    pl.kernel,
    out_type=jax.ShapeDtypeStruct((NC * NP,), jnp.float32),
    mesh=_mesh,
    compiler_params=_params,
    scratch_types=[
        pltpu.VMEM((N,), jnp.float32),
        pltpu.VMEM((EP,), jnp.int32),
        pltpu.VMEM((EP,), jnp.int32),
        pltpu.VMEM((NP,), jnp.float32),
        pltpu.VMEM_SHARED((NS, NP), jnp.float32),
        pltpu.VMEM((RPT,), jnp.float32),
        pltpu.VMEM((RPT,), jnp.float32),
        pltpu.SemaphoreType.DMA,
    ],
)
def _agg2_kernel(hs_hbm, src_hbm, dst_hbm, out_hbm, hv, sv, dv, acc_v,
                 shared, red, tmp, sem):
    del sem
    cid = lax.axis_index("c")
    sid = lax.axis_index("s")
    wid = sid * NC + cid
    pltpu.sync_copy(hs_hbm, hv)
    pltpu.sync_copy(src_hbm.at[wid], sv)
    pltpu.sync_copy(dst_hbm.at[wid], dv)
    _zero_1d(acc_v, NP)

    def gbody(j, carry):
        for u in range(U):
            s16 = sv[pl.ds((j * U + u) * 16, 16)]
            d16 = dv[pl.ds((j * U + u) * 16, 16)]
            vals = plsc.load_gather(hv, [s16])
            plsc.addupdate_scatter(acc_v, [d16], vals)
        return carry

    lax.fori_loop(0, EP // (16 * U), gbody, 0)
    _spmem_reduce_write(acc_v, shared, red, tmp, out_hbm, cid, sid)


_R = 2000


def _tc1_body(x_ref, w_ref, p_ref, hs_ref, dis_ref):
    deg = 1.0 + p_ref[0] + p_ref[1]
    dis = lax.rsqrt(deg)
    h = jnp.dot(x_ref[...], w_ref[...], preferred_element_type=jnp.float32)
    hs_ref[...] = h * dis
    dis_ref[...] = dis


def _tc2_body(p_ref, hs_ref, dis_ref, w2_ref, b1_ref, hs2_ref):
    agg = p_ref[0] + p_ref[1] + hs_ref[...]
    dis = dis_ref[...]
    out1 = agg * dis + b1_ref[...]
    r = jnp.maximum(out1, 0.0)
    h2 = jnp.dot(r, w2_ref[...], preferred_element_type=jnp.float32)
    hs2_ref[...] = h2 * dis


def _tc3_body(p_ref, hs2_ref, dis_ref, b2_ref, out_ref):
    agg = p_ref[0] + p_ref[1]
    out_ref[...] = dis_ref[...] * (agg + hs2_ref[...]) + b2_ref[...]


def kernel(x, edge_index, W1, b1, W2, b2):
    src = edge_index[0]
    dst = edge_index[1]
    src_w = src.reshape(NW, EP)
    dst_w = dst.reshape(NW, EP)

    padn = EPP - EP
    pad_src = (jnp.arange(padn, dtype=jnp.int32) * 41 + 7) % N
    pad_dst = N + jnp.arange(padn, dtype=jnp.int32) % (NP - N)
    src3 = jnp.concatenate(
        [src_w, jnp.broadcast_to(pad_src, (NW, padn))], axis=1
    ).reshape(NW, K1, CH)
    dst3 = jnp.concatenate(
        [dst_w, jnp.broadcast_to(pad_dst, (NW, padn))], axis=1
    ).reshape(NW, K1, CH)

    degp = _deg_kernel(dst_w).reshape(NC, NP, 1)

    hs1, dis = pl.pallas_call(
        _tc1_body,
        grid=(N // _R,),
        in_specs=[
            pl.BlockSpec((_R, D), lambda i: (i, 0)),
            pl.BlockSpec((D, H), lambda i: (0, 0)),
            pl.BlockSpec((NC, _R, 1), lambda i: (0, i, 0)),
        ],
        out_specs=[
            pl.BlockSpec((_R, H), lambda i: (i, 0)),
            pl.BlockSpec((_R, 1), lambda i: (i, 0)),
        ],
        out_shape=[
            jax.ShapeDtypeStruct((N, H), jnp.float32),
            jax.ShapeDtypeStruct((N, 1), jnp.float32),
        ],
    )(x, W1, degp)

    part1 = _agg1_kernel(hs1, src3, dst3)

    hs2 = pl.pallas_call(
        _tc2_body,
        grid=(N // _R,),
        in_specs=[
            pl.BlockSpec((NC, _R, H), lambda i: (0, i, 0)),
            pl.BlockSpec((_R, H), lambda i: (i, 0)),
            pl.BlockSpec((_R, 1), lambda i: (i, 0)),
            pl.BlockSpec((H, 1), lambda i: (0, 0)),
            pl.BlockSpec((1, H), lambda i: (0, 0)),
        ],
        out_specs=pl.BlockSpec((_R, 1), lambda i: (i, 0)),
        out_shape=jax.ShapeDtypeStruct((N, 1), jnp.float32),
    )(part1, hs1, dis, W2, b1.reshape(1, H))

    part2 = _agg2_kernel(hs2.reshape(N), src_w, dst_w).reshape(NC, NP, 1)

    out = pl.pallas_call(
        _tc3_body,
        grid=(N // _R,),
        in_specs=[
            pl.BlockSpec((NC, _R, 1), lambda i: (0, i, 0)),
            pl.BlockSpec((_R, 1), lambda i: (i, 0)),
            pl.BlockSpec((_R, 1), lambda i: (i, 0)),
            pl.BlockSpec((1, 1), lambda i: (0, 0)),
        ],
        out_specs=pl.BlockSpec((_R, 1), lambda i: (i, 0)),
        out_shape=jax.ShapeDtypeStruct((N, 1), jnp.float32),
    )(part2, hs2, dis, b2.reshape(1, 1))

    return out


# speedup vs baseline: 1.2912x; 1.2912x over previous
import functools

import jax
import jax.numpy as jnp
from jax import lax
from jax.experimental import pallas as pl
from jax.experimental.pallas import tpu as pltpu
from jax.experimental.pallas import tpu_sc as plsc

N = 10000
NP = 10240
D = 128
H = 16
NC = 2
NS = 16
NW = NC * NS
E = 320000
EP = E // NW
CH = 128
G = 8
K1 = 80
NG = K1 // G
EPP = K1 * CH
RPT = NP // NS
U = 5

_mesh = plsc.VectorSubcoreMesh(core_axis_name="c", subcore_axis_name="s")
_params = pltpu.CompilerParams(needs_layout_passes=False,
                               use_tc_tiling_on_sc=False)


def _zero_1d(ref, n):
    def zbody(i, carry):
        ref[pl.ds(i * 16, 16)] = jnp.zeros((16,), jnp.float32)
        return carry

    lax.fori_loop(0, n // 16, zbody, 0)


def _spmem_reduce_write(acc_v, shared, red, tmp, out0, out1, cid, sid):
    pltpu.sync_copy(acc_v, shared.at[sid])
    plsc.subcore_barrier()
    sl = pl.ds(sid * RPT, RPT)
    pltpu.sync_copy(shared.at[0].at[sl], red)
    for t in range(1, NS):
        pltpu.sync_copy(shared.at[t].at[sl], tmp)

        def abody(i, c2):
            red[pl.ds(i * 16, 16)] = (red[pl.ds(i * 16, 16)]
                                      + tmp[pl.ds(i * 16, 16)])
            return c2

        lax.fori_loop(0, RPT // 16, abody, 0)

    @pl.when(cid == 0)
    def _w0():
        pltpu.sync_copy(red, out0.at[sl])

    @pl.when(cid == 1)
    def _w1():
        pltpu.sync_copy(red, out1.at[sl])


@functools.partial(
    pl.kernel,
    out_type=[jax.ShapeDtypeStruct((NP,), jnp.float32),
              jax.ShapeDtypeStruct((NP,), jnp.float32)],
    mesh=_mesh,
    compiler_params=_params,
    scratch_types=[
        pltpu.VMEM((EP,), jnp.int32),
        pltpu.VMEM((NP,), jnp.float32),
        pltpu.VMEM_SHARED((NS, NP), jnp.float32),
        pltpu.VMEM((RPT,), jnp.float32),
        pltpu.VMEM((RPT,), jnp.float32),
        pltpu.SemaphoreType.DMA,
    ],
)
def _deg_kernel(edge_hbm, out0, out1, dst_v, acc_v, shared, red, tmp, sem):
    del sem
    cid = lax.axis_index("c")
    sid = lax.axis_index("s")
    wid = sid * NC + cid
    pltpu.sync_copy(edge_hbm.at[1].at[pl.ds(wid * EP, EP)], dst_v)
    _zero_1d(acc_v, NP)
    ones = jnp.ones((16,), jnp.float32)

    def cbody(j, carry):
        for u in range(U):
            d16 = dst_v[pl.ds((j * U + u) * 16, 16)]
            plsc.addupdate_scatter(acc_v, [d16], ones)
        return carry

    lax.fori_loop(0, EP // (16 * U), cbody, 0)
    _spmem_reduce_write(acc_v, shared, red, tmp, out0, out1, cid, sid)


@functools.partial(
    pl.kernel,
    out_type=jax.ShapeDtypeStruct((NC, NP, H), jnp.float32),
    mesh=_mesh,
    compiler_params=_params,
    scratch_types=[
        pltpu.VMEM((K1, CH), jnp.int32),
        pltpu.VMEM((K1, CH), jnp.int32),
        pltpu.VMEM((3, G, CH, H), jnp.float32),
        pltpu.VMEM((RPT, H), jnp.float32),
        pltpu.VMEM_SHARED((NP, H), jnp.float32),
        pltpu.SemaphoreType.DMA,
        pltpu.SemaphoreType.DMA,
    ],
)
def _agg1_kernel(hs_hbm, epad_hbm, out_hbm, si, di, rows, zbuf,
                 acc_sh, sem_g, sem_s):
    cid = lax.axis_index("c")
    sid = lax.axis_index("s")
    wid = sid * NC + cid
    pltpu.sync_copy(epad_hbm.at[0].at[wid], si)
    pltpu.sync_copy(epad_hbm.at[1].at[wid], di)

    def zb(i, carry):
        zbuf[i, :] = jnp.zeros((H,), jnp.float32)
        return carry

    lax.fori_loop(0, RPT, zb, 0)
    pltpu.sync_copy(zbuf, acc_sh.at[pl.ds(sid * RPT, RPT)])
    plsc.subcore_barrier()

    for b in range(G):
        pltpu.async_copy(hs_hbm.at[si.at[b]], rows.at[0, b], sem_g)

    def body(g, carry):
        b_cur = lax.rem(g, 3)
        b_nxt = lax.rem(g + 1, 3)

        @pl.when(g >= 2)
        def _drain():
            for b in range(G):
                pltpu.make_async_copy(rows.at[b_nxt, b],
                                      acc_sh.at[di.at[(g - 2) * G + b]],
                                      sem_s).wait()

        @pl.when(g + 1 < NG)
        def _fire():
            for b in range(G):
                pltpu.async_copy(hs_hbm.at[si.at[(g + 1) * G + b]],
                                 rows.at[b_nxt, b], sem_g)

        for b in range(G):
            pltpu.make_async_copy(hs_hbm.at[si.at[g * G + b]],
                                  rows.at[b_cur, b], sem_g).wait()
        for b in range(G):
            pltpu.async_copy(rows.at[b_cur, b], acc_sh.at[di.at[g * G + b]],
                             sem_s, add=True)
        return carry

    lax.fori_loop(0, NG, body, 0)
    for g in (NG - 2, NG - 1):
        for b in range(G):
            pltpu.make_async_copy(rows.at[g % 3, b],
                                  acc_sh.at[di.at[g * G + b]], sem_s).wait()
    plsc.subcore_barrier()
    sl = pl.ds(sid * RPT, RPT)
    pltpu.sync_copy(acc_sh.at[sl], out_hbm.at[cid].at[sl])


@functools.partial(
    pl.kernel,
    out_type=[jax.ShapeDtypeStruct((NP,), jnp.float32),
              jax.ShapeDtypeStruct((NP,), jnp.float32)],
    mesh=_mesh,
    compiler_params=_params,
    scratch_types=[
        pltpu.VMEM((NP,), jnp.float32),
        pltpu.VMEM((EP,), jnp.int32),
        pltpu.VMEM((EP,), jnp.int32),
        pltpu.VMEM((NP,), jnp.float32),
        pltpu.VMEM_SHARED((NS, NP), jnp.float32),
        pltpu.VMEM((RPT,), jnp.float32),
        pltpu.VMEM((RPT,), jnp.float32),
        pltpu.SemaphoreType.DMA,
    ],
)
def _agg2_kernel(hs_hbm, edge_hbm, out0, out1, hv, sv, dv, acc_v,
                 shared, red, tmp, sem):
    del sem
    cid = lax.axis_index("c")
    sid = lax.axis_index("s")
    wid = sid * NC + cid
    pltpu.sync_copy(hs_hbm, hv)
    pltpu.sync_copy(edge_hbm.at[0].at[pl.ds(wid * EP, EP)], sv)
    pltpu.sync_copy(edge_hbm.at[1].at[pl.ds(wid * EP, EP)], dv)
    _zero_1d(acc_v, NP)

    def gbody(j, carry):
        for u in range(U):
            s16 = sv[pl.ds((j * U + u) * 16, 16)]
            d16 = dv[pl.ds((j * U + u) * 16, 16)]
            vals = plsc.load_gather(hv, [s16])
            plsc.addupdate_scatter(acc_v, [d16], vals)
        return carry

    lax.fori_loop(0, EP // (16 * U), gbody, 0)
    _spmem_reduce_write(acc_v, shared, red, tmp, out0, out1, cid, sid)


_R = 2048


def _tc1_body(x_ref, w_ref, p0_ref, p1_ref, hs_ref, dis_ref):
    i = pl.program_id(0)
    deg = 1.0 + p0_ref[pl.ds(i * _R, _R)] + p1_ref[pl.ds(i * _R, _R)]
    dis = lax.rsqrt(deg)[:, None]
    h = jnp.dot(x_ref[...], w_ref[...], preferred_element_type=jnp.float32)
    hs_ref[...] = h * dis
    dis_ref[...] = dis


def _tc2_body(p_ref, hs_ref, dis_ref, w2_ref, b1_ref, hs2_ref):
    agg = p_ref[0] + p_ref[1] + hs_ref[...]
    dis = dis_ref[...]
    out1 = agg * dis + b1_ref[...]
    r = jnp.maximum(out1, 0.0)
    h2 = jnp.dot(r, w2_ref[...], preferred_element_type=jnp.float32)
    hs2_ref[...] = h2 * dis


def _tc3_body(p0_ref, p1_ref, hs2_ref, dis_ref, b2_ref, out_ref):
    i = pl.program_id(0)
    agg = (p0_ref[pl.ds(i * _R, _R)] + p1_ref[pl.ds(i * _R, _R)])[:, None]
    out_ref[...] = dis_ref[...] * (agg + hs2_ref[...]) + b2_ref[...]


def kernel(x, edge_index, W1, b1, W2, b2):
    padn = EPP - EP
    pad_src = (jnp.arange(padn, dtype=jnp.int32) * 41 + 7) % N
    pad_dst = N + jnp.arange(padn, dtype=jnp.int32) % (NP - N)
    pads = jnp.stack([pad_src, pad_dst])
    epad = jnp.concatenate(
        [edge_index.reshape(2, NW, EP),
         jnp.broadcast_to(pads[:, None, :], (2, NW, padn))], axis=2
    ).reshape(2, NW, K1, CH)

    degp0, degp1 = _deg_kernel(edge_index)
    xp = jnp.pad(x, ((0, NP - N), (0, 0)))

    hs1, dis = pl.pallas_call(
        _tc1_body,
        grid=(NP // _R,),
        in_specs=[
            pl.BlockSpec((_R, D), lambda i: (i, 0)),
            pl.BlockSpec((D, H), lambda i: (0, 0)),
            pl.BlockSpec((NP,), lambda i: (0,)),
            pl.BlockSpec((NP,), lambda i: (0,)),
        ],
        out_specs=[
            pl.BlockSpec((_R, H), lambda i: (i, 0)),
            pl.BlockSpec((_R, 1), lambda i: (i, 0)),
        ],
        out_shape=[
            jax.ShapeDtypeStruct((NP, H), jnp.float32),
            jax.ShapeDtypeStruct((NP, 1), jnp.float32),
        ],
    )(xp, W1, degp0, degp1)

    part1 = _agg1_kernel(hs1, epad)

    hs2 = pl.pallas_call(
        _tc2_body,
        grid=(NP // _R,),
        in_specs=[
            pl.BlockSpec((NC, _R, H), lambda i: (0, i, 0)),
            pl.BlockSpec((_R, H), lambda i: (i, 0)),
            pl.BlockSpec((_R, 1), lambda i: (i, 0)),
            pl.BlockSpec((H, 1), lambda i: (0, 0)),
            pl.BlockSpec((1, H), lambda i: (0, 0)),
        ],
        out_specs=pl.BlockSpec((_R, 1), lambda i: (i, 0)),
        out_shape=jax.ShapeDtypeStruct((NP, 1), jnp.float32),
    )(part1, hs1, dis, W2, b1.reshape(1, H))

    p20, p21 = _agg2_kernel(hs2.reshape(NP), edge_index)

    out = pl.pallas_call(
        _tc3_body,
        grid=(NP // _R,),
        in_specs=[
            pl.BlockSpec((NP,), lambda i: (0,)),
            pl.BlockSpec((NP,), lambda i: (0,)),
            pl.BlockSpec((_R, 1), lambda i: (i, 0)),
            pl.BlockSpec((_R, 1), lambda i: (i, 0)),
            pl.BlockSpec((1, 1), lambda i: (0, 0)),
        ],
        out_specs=pl.BlockSpec((_R, 1), lambda i: (i, 0)),
        out_shape=jax.ShapeDtypeStruct((NP, 1), jnp.float32),
    )(p20, p21, hs2, dis, b2.reshape(1, 1))

    return out[:N]


# speedup vs baseline: 1.3340x; 1.0331x over previous
import functools

import jax
import jax.numpy as jnp
from jax import lax
from jax.experimental import pallas as pl
from jax.experimental.pallas import tpu as pltpu
from jax.experimental.pallas import tpu_sc as plsc

N = 10000
NP = 10240
D = 128
H = 16
NC = 2
NS = 16
NW = NC * NS
E = 320000
EP = E // NW
CH = 128
G = 8
K1 = 80
NG = K1 // G
EPP = K1 * CH
RPT = NP // NS
U = 5

_mesh = plsc.VectorSubcoreMesh(core_axis_name="c", subcore_axis_name="s")
_params = pltpu.CompilerParams(needs_layout_passes=False,
                               use_tc_tiling_on_sc=False)


def _zero_1d(ref, n):
    def zbody(i, carry):
        ref[pl.ds(i * 16, 16)] = jnp.zeros((16,), jnp.float32)
        return carry

    lax.fori_loop(0, n // 16, zbody, 0)


def _spmem_reduce_write(acc_v, shared, red, tmp, out0, out1, cid, sid):
    pltpu.sync_copy(acc_v, shared.at[sid])
    plsc.subcore_barrier()
    sl = pl.ds(sid * RPT, RPT)
    pltpu.sync_copy(shared.at[0].at[sl], red)
    for t in range(1, NS):
        pltpu.sync_copy(shared.at[t].at[sl], tmp)

        def abody(i, c2):
            red[pl.ds(i * 16, 16)] = (red[pl.ds(i * 16, 16)]
                                      + tmp[pl.ds(i * 16, 16)])
            return c2

        lax.fori_loop(0, RPT // 16, abody, 0)

    @pl.when(cid == 0)
    def _w0():
        pltpu.sync_copy(red, out0.at[sl])

    @pl.when(cid == 1)
    def _w1():
        pltpu.sync_copy(red, out1.at[sl])


@functools.partial(
    pl.kernel,
    out_type=[jax.ShapeDtypeStruct((NP,), jnp.float32),
              jax.ShapeDtypeStruct((NP,), jnp.float32)],
    mesh=_mesh,
    compiler_params=_params,
    scratch_types=[
        pltpu.VMEM((EP,), jnp.int32),
        pltpu.VMEM((NP,), jnp.float32),
        pltpu.VMEM_SHARED((NS, NP), jnp.float32),
        pltpu.VMEM((RPT,), jnp.float32),
        pltpu.VMEM((RPT,), jnp.float32),
        pltpu.SemaphoreType.DMA,
    ],
)
def _deg_kernel(edge_hbm, out0, out1, dst_v, acc_v, shared, red, tmp, sem):
    del sem
    cid = lax.axis_index("c")
    sid = lax.axis_index("s")
    wid = sid * NC + cid
    pltpu.sync_copy(edge_hbm.at[1].at[pl.ds(wid * EP, EP)], dst_v)
    _zero_1d(acc_v, NP)
    ones = jnp.ones((16,), jnp.float32)

    def cbody(j, carry):
        for u in range(U):
            d16 = dst_v[pl.ds((j * U + u) * 16, 16)]
            plsc.addupdate_scatter(acc_v, [d16], ones)
        return carry

    lax.fori_loop(0, EP // (16 * U), cbody, 0)
    _spmem_reduce_write(acc_v, shared, red, tmp, out0, out1, cid, sid)


@functools.partial(
    pl.kernel,
    out_type=jax.ShapeDtypeStruct((NC, NP, H), jnp.float32),
    mesh=_mesh,
    compiler_params=_params,
    scratch_types=[
        pltpu.VMEM((K1, CH), jnp.int32),
        pltpu.VMEM((K1, CH), jnp.int32),
        pltpu.VMEM((3, G, CH, H), jnp.float32),
        pltpu.VMEM((RPT, H), jnp.float32),
        pltpu.VMEM_SHARED((NP, H), jnp.float32),
        pltpu.SemaphoreType.DMA,
        pltpu.SemaphoreType.DMA,
    ],
)
def _agg1_kernel(hs_hbm, epad_hbm, out_hbm, si, di, rows, zbuf,
                 acc_sh, sem_g, sem_s):
    cid = lax.axis_index("c")
    sid = lax.axis_index("s")
    wid = sid * NC + cid
    pltpu.sync_copy(epad_hbm.at[0].at[wid], si)
    pltpu.sync_copy(epad_hbm.at[1].at[wid], di)

    def zb(i, carry):
        zbuf[i, :] = jnp.zeros((H,), jnp.float32)
        return carry

    lax.fori_loop(0, RPT, zb, 0)
    pltpu.sync_copy(zbuf, acc_sh.at[pl.ds(sid * RPT, RPT)])
    plsc.subcore_barrier()

    for b in range(G):
        pltpu.async_copy(hs_hbm.at[si.at[b]], rows.at[0, b], sem_g)

    def body(g, carry):
        b_cur = lax.rem(g, 3)
        b_nxt = lax.rem(g + 1, 3)

        @pl.when(g >= 2)
        def _drain():
            for b in range(G):
                pltpu.make_async_copy(rows.at[b_nxt, b],
                                      acc_sh.at[di.at[(g - 2) * G + b]],
                                      sem_s).wait()

        @pl.when(g + 1 < NG)
        def _fire():
            for b in range(G):
                pltpu.async_copy(hs_hbm.at[si.at[(g + 1) * G + b]],
                                 rows.at[b_nxt, b], sem_g)

        for b in range(G):
            pltpu.make_async_copy(hs_hbm.at[si.at[g * G + b]],
                                  rows.at[b_cur, b], sem_g).wait()
        for b in range(G):
            pltpu.async_copy(rows.at[b_cur, b], acc_sh.at[di.at[g * G + b]],
                             sem_s, add=True)
        return carry

    lax.fori_loop(0, NG, body, 0)
    for g in (NG - 2, NG - 1):
        for b in range(G):
            pltpu.make_async_copy(rows.at[g % 3, b],
                                  acc_sh.at[di.at[g * G + b]], sem_s).wait()
    plsc.subcore_barrier()
    sl = pl.ds(sid * RPT, RPT)
    pltpu.sync_copy(acc_sh.at[sl], out_hbm.at[cid].at[sl])


@functools.partial(
    pl.kernel,
    out_type=[jax.ShapeDtypeStruct((NP,), jnp.float32),
              jax.ShapeDtypeStruct((NP,), jnp.float32)],
    mesh=_mesh,
    compiler_params=_params,
    scratch_types=[
        pltpu.VMEM((NP,), jnp.float32),
        pltpu.VMEM((EP,), jnp.int32),
        pltpu.VMEM((EP,), jnp.int32),
        pltpu.VMEM((NP,), jnp.float32),
        pltpu.VMEM_SHARED((NS, NP), jnp.float32),
        pltpu.VMEM((RPT,), jnp.float32),
        pltpu.VMEM((RPT,), jnp.float32),
        pltpu.SemaphoreType.DMA,
    ],
)
def _agg2_kernel(hs_hbm, edge_hbm, out0, out1, hv, sv, dv, acc_v,
                 shared, red, tmp, sem):
    del sem
    cid = lax.axis_index("c")
    sid = lax.axis_index("s")
    wid = sid * NC + cid
    pltpu.sync_copy(hs_hbm, hv)
    pltpu.sync_copy(edge_hbm.at[0].at[pl.ds(wid * EP, EP)], sv)
    pltpu.sync_copy(edge_hbm.at[1].at[pl.ds(wid * EP, EP)], dv)
    _zero_1d(acc_v, NP)

    def gbody(j, carry):
        for u in range(U):
            s16 = sv[pl.ds((j * U + u) * 16, 16)]
            d16 = dv[pl.ds((j * U + u) * 16, 16)]
            vals = plsc.load_gather(hv, [s16])
            plsc.addupdate_scatter(acc_v, [d16], vals)
        return carry

    lax.fori_loop(0, EP // (16 * U), gbody, 0)
    _spmem_reduce_write(acc_v, shared, red, tmp, out0, out1, cid, sid)


_R = 2048


def _tc1_body(x_ref, w_ref, p0_ref, p1_ref, hs_ref, dis_ref):
    i = pl.program_id(0)
    deg = 1.0 + p0_ref[pl.ds(i * _R, _R)] + p1_ref[pl.ds(i * _R, _R)]
    dis = lax.rsqrt(deg)[:, None]
    h = jnp.dot(x_ref[...], w_ref[...], preferred_element_type=jnp.float32)
    hs_ref[...] = h * dis
    dis_ref[...] = dis


def _tc2_body(p_ref, hs_ref, dis_ref, w2_ref, b1_ref, hs2_ref, hs2f_ref):
    agg = p_ref[0] + p_ref[1] + hs_ref[...]
    dis = dis_ref[...]
    out1 = agg * dis + b1_ref[...]
    r = jnp.maximum(out1, 0.0)
    h2 = jnp.dot(r, w2_ref[...], preferred_element_type=jnp.float32)
    h2d = h2 * dis
    hs2_ref[...] = h2d
    hs2f_ref[...] = h2d[:, 0]


def _tc3_body(p0_ref, p1_ref, hs2_ref, dis_ref, b2_ref, out_ref):
    i = pl.program_id(0)
    agg = (p0_ref[pl.ds(i * _R, _R)] + p1_ref[pl.ds(i * _R, _R)])[:, None]
    out_ref[...] = dis_ref[...] * (agg + hs2_ref[...]) + b2_ref[...]


def kernel(x, edge_index, W1, b1, W2, b2):
    padn = EPP - EP
    pad_src = (jnp.arange(padn, dtype=jnp.int32) * 41 + 7) % N
    pad_dst = N + jnp.arange(padn, dtype=jnp.int32) % (NP - N)
    pads = jnp.stack([pad_src, pad_dst])
    epad = jnp.concatenate(
        [edge_index.reshape(2, NW, EP),
         jnp.broadcast_to(pads[:, None, :], (2, NW, padn))], axis=2
    ).reshape(2, NW, K1, CH)

    degp0, degp1 = _deg_kernel(edge_index)

    hs1, dis = pl.pallas_call(
        _tc1_body,
        grid=(NP // _R,),
        in_specs=[
            pl.BlockSpec((_R, D), lambda i: (i, 0)),
            pl.BlockSpec((D, H), lambda i: (0, 0)),
            pl.BlockSpec((NP,), lambda i: (0,)),
            pl.BlockSpec((NP,), lambda i: (0,)),
        ],
        out_specs=[
            pl.BlockSpec((_R, H), lambda i: (i, 0)),
            pl.BlockSpec((_R, 1), lambda i: (i, 0)),
        ],
        out_shape=[
            jax.ShapeDtypeStruct((NP, H), jnp.float32),
            jax.ShapeDtypeStruct((NP, 1), jnp.float32),
        ],
    )(x, W1, degp0, degp1)

    part1 = _agg1_kernel(hs1, epad)

    hs2 = pl.pallas_call(
        _tc2_body,
        grid=(NP // _R,),
        in_specs=[
            pl.BlockSpec((NC, _R, H), lambda i: (0, i, 0)),
            pl.BlockSpec((_R, H), lambda i: (i, 0)),
            pl.BlockSpec((_R, 1), lambda i: (i, 0)),
            pl.BlockSpec((H, 1), lambda i: (0, 0)),
            pl.BlockSpec((1, H), lambda i: (0, 0)),
        ],
        out_specs=[pl.BlockSpec((_R, 1), lambda i: (i, 0)),
                   pl.BlockSpec((_R,), lambda i: (i,))],
        out_shape=[jax.ShapeDtypeStruct((NP, 1), jnp.float32),
                   jax.ShapeDtypeStruct((NP,), jnp.float32)],
    )(part1, hs1, dis, W2, b1.reshape(1, H))
    hs2, hs2f = hs2

    p20, p21 = _agg2_kernel(hs2f, edge_index)

    out = pl.pallas_call(
        _tc3_body,
        grid=(NP // _R,),
        in_specs=[
            pl.BlockSpec((NP,), lambda i: (0,)),
            pl.BlockSpec((NP,), lambda i: (0,)),
            pl.BlockSpec((_R, 1), lambda i: (i, 0)),
            pl.BlockSpec((_R, 1), lambda i: (i, 0)),
            pl.BlockSpec((1, 1), lambda i: (0, 0)),
        ],
        out_specs=pl.BlockSpec((_R, 1), lambda i: (i, 0)),
        out_shape=jax.ShapeDtypeStruct((N, 1), jnp.float32),
    )(p20, p21, hs2, dis, b2.reshape(1, 1))

    return out
